# Initial kernel scaffold; baseline (speedup 1.0000x reference)
import jax
import jax.numpy as jnp
from jax.experimental import pallas as pl


def kernel(atomic_numbers, embedding):
    raise NotImplementedError("write your pallas kernel here")


# speedup vs baseline: 1.0889x; 1.0889x over previous
import functools

import jax
import jax.numpy as jnp
from jax import lax
from jax.experimental import pallas as pl
from jax.experimental.pallas import tpu as pltpu
from jax.experimental.pallas import tpu_sc as plsc

N_NODES = 100000
TOTAL_DIM = 128
CHUNK = 80
NUM_CHUNKS = N_NODES // CHUNK
NUM_WORKERS = 32
CHUNKS_PER_WORKER = -(-NUM_CHUNKS // NUM_WORKERS)

_mesh = plsc.VectorSubcoreMesh(core_axis_name="c", subcore_axis_name="s")


@functools.partial(
    pl.kernel,
    mesh=_mesh,
    out_type=jax.ShapeDtypeStruct((N_NODES, TOTAL_DIM), jnp.float32),
    scratch_types=[
        pltpu.VMEM((CHUNK,), jnp.int32),
        pltpu.VMEM((CHUNK, TOTAL_DIM), jnp.float32),
        pltpu.SemaphoreType.DMA,
    ],
)
def _gather_kernel(idx_hbm, table_hbm, out_hbm, idx_v, rows_v, sem):
    wid = lax.axis_index("s") * 2 + lax.axis_index("c")

    def body(j, carry):
        cid = wid + j * NUM_WORKERS

        @pl.when(cid < NUM_CHUNKS)
        def _():
            base = cid * CHUNK
            pltpu.sync_copy(idx_hbm.at[pl.ds(base, CHUNK)], idx_v)
            pltpu.async_copy(table_hbm.at[idx_v], rows_v, sem).wait()
            pltpu.sync_copy(rows_v, out_hbm.at[pl.ds(base, CHUNK)])

        return carry

    lax.fori_loop(0, CHUNKS_PER_WORKER, body, 0)


def kernel(atomic_numbers, embedding):
    idx = atomic_numbers.astype(jnp.int32)
    return _gather_kernel(idx, embedding)


# speedup vs baseline: 1.8087x; 1.6610x over previous
import functools

import jax
import jax.numpy as jnp
from jax import lax
from jax.experimental import pallas as pl
from jax.experimental.pallas import tpu as pltpu
from jax.experimental.pallas import tpu_sc as plsc

N_NODES = 100000
TOTAL_DIM = 128
CHUNK = 400
NUM_CHUNKS = N_NODES // CHUNK
NUM_WORKERS = 32
CHUNKS_PER_WORKER = -(-NUM_CHUNKS // NUM_WORKERS)

_mesh = plsc.VectorSubcoreMesh(core_axis_name="c", subcore_axis_name="s")


@functools.partial(
    pl.kernel,
    mesh=_mesh,
    out_type=jax.ShapeDtypeStruct((N_NODES, TOTAL_DIM), jnp.float32),
    scratch_types=[
        pltpu.VMEM((CHUNK,), jnp.int32),
        pltpu.VMEM((CHUNK, TOTAL_DIM), jnp.float32),
        pltpu.SemaphoreType.DMA,
    ],
)
def _gather_kernel(idx_hbm, table_hbm, out_hbm, idx_v, rows_v, sem):
    wid = lax.axis_index("s") * 2 + lax.axis_index("c")

    def body(j, carry):
        cid = wid + j * NUM_WORKERS

        @pl.when(cid < NUM_CHUNKS)
        def _():
            base = cid * CHUNK
            pltpu.sync_copy(idx_hbm.at[pl.ds(base, CHUNK)], idx_v)
            pltpu.async_copy(table_hbm.at[idx_v], rows_v, sem).wait()
            pltpu.sync_copy(rows_v, out_hbm.at[pl.ds(base, CHUNK)])

        return carry

    lax.fori_loop(0, CHUNKS_PER_WORKER, body, 0)


def kernel(atomic_numbers, embedding):
    idx = atomic_numbers.astype(jnp.int32)
    return _gather_kernel(idx, embedding)


# speedup vs baseline: 1.9886x; 1.0995x over previous
import functools

import jax
import jax.numpy as jnp
from jax import lax
from jax.experimental import pallas as pl
from jax.experimental.pallas import tpu as pltpu
from jax.experimental.pallas import tpu_sc as plsc

N_NODES = 100000
TOTAL_DIM = 128
CHUNK = 800
NUM_CHUNKS = N_NODES // CHUNK
NUM_WORKERS = 32
CHUNKS_PER_WORKER = -(-NUM_CHUNKS // NUM_WORKERS)

_mesh = plsc.VectorSubcoreMesh(core_axis_name="c", subcore_axis_name="s")


@functools.partial(
    pl.kernel,
    mesh=_mesh,
    out_type=jax.ShapeDtypeStruct((N_NODES, TOTAL_DIM), jnp.float32),
    scratch_types=[
        pltpu.VMEM((CHUNK,), jnp.int32),
        pltpu.VMEM((CHUNK, TOTAL_DIM), jnp.float32),
        pltpu.SemaphoreType.DMA,
    ],
)
def _gather_kernel(idx_hbm, table_hbm, out_hbm, idx_v, rows_v, sem):
    wid = lax.axis_index("s") * 2 + lax.axis_index("c")

    def body(j, carry):
        cid = wid + j * NUM_WORKERS

        @pl.when(cid < NUM_CHUNKS)
        def _():
            base = cid * CHUNK
            pltpu.sync_copy(idx_hbm.at[pl.ds(base, CHUNK)], idx_v)
            pltpu.async_copy(table_hbm.at[idx_v], rows_v, sem).wait()
            pltpu.sync_copy(rows_v, out_hbm.at[pl.ds(base, CHUNK)])

        return carry

    lax.fori_loop(0, CHUNKS_PER_WORKER, body, 0)


def kernel(atomic_numbers, embedding):
    idx = atomic_numbers.astype(jnp.int32)
    return _gather_kernel(idx, embedding)
